# baseline (speedup 1.0000x reference)
import functools

import jax
import jax.numpy as jnp
from jax import lax
from jax.experimental import pallas as pl
from jax.experimental.pallas import tpu as pltpu
from jax.experimental.pallas import tpu_sc as plsc

_B, _N, _C, _M = 4, 100000, 128, 25000
_ROWS = 256
_NCH = (_M + _ROWS - 1) // _ROWS
_TAIL = _M - (_NCH - 1) * _ROWS
_TASKS = _B * _NCH
_NSUB = 16
_FA = _TASKS // 2
_KMAX = (max(_FA, _TASKS - _FA) + _NSUB - 1) // _NSUB
_NBUF = 3


def _sc_gather(table, idx):
    mesh = plsc.VectorSubcoreMesh(core_axis_name="c", subcore_axis_name="s")

    @functools.partial(
        pl.kernel,
        mesh=mesh,
        out_type=jax.ShapeDtypeStruct((_B * _M, _C), jnp.float32),
        scratch_types=[
            pltpu.VMEM((_KMAX * _ROWS,), jnp.int32),
            pltpu.VMEM((_NBUF, _ROWS, _C), jnp.float32),
            pltpu.SemaphoreType.DMA,
            pltpu.SemaphoreType.DMA,
            pltpu.SemaphoreType.DMA,
            pltpu.SemaphoreType.DMA,
            pltpu.SemaphoreType.DMA,
            pltpu.SemaphoreType.DMA,
            pltpu.SemaphoreType.DMA,
            pltpu.SemaphoreType.DMA,
            pltpu.SemaphoreType.DMA,
            pltpu.SemaphoreType.DMA,
        ],
    )
    def k(table_hbm, idx_hbm, out_hbm, idx_v, rows_v,
          i0, i1, i2, i3, g0, g1, g2, w0, w1, w2):
        cid = lax.axis_index("c")
        sid = lax.axis_index("s")
        base = cid * _FA
        limit = _FA + cid * (_TASKS - _FA)
        sem_i = (i0, i1, i2, i3)
        sem_g = (g0, g1, g2)
        sem_w = (w0, w1, w2)

        def task(kk):
            return base + sid + kk * _NSUB

        def task_parts(t):
            return t // _NCH, t % _NCH

        def do_idx(kk, t, start):
            _, ch = task_parts(t)

            @pl.when(ch < _NCH - 1)
            def _():
                cp = pltpu.make_async_copy(
                    idx_hbm.at[pl.ds(ch * _ROWS, _ROWS)],
                    idx_v.at[pl.ds(kk * _ROWS, _ROWS)], sem_i[kk % 4])
                cp.start() if start else cp.wait()

            @pl.when(ch == _NCH - 1)
            def _():
                cp = pltpu.make_async_copy(
                    idx_hbm.at[pl.ds((_NCH - 1) * _ROWS, _TAIL)],
                    idx_v.at[pl.ds(kk * _ROWS, _TAIL)], sem_i[kk % 4])
                cp.start() if start else cp.wait()

        def do_gathers(kk, t, start):
            ib = kk % _NBUF
            b, ch = task_parts(t)

            def one(nrows):
                cp = pltpu.make_async_copy(
                    table_hbm.at[b].at[idx_v.at[pl.ds(kk * _ROWS, nrows)]],
                    rows_v.at[ib].at[pl.ds(0, nrows)],
                    sem_g[ib])
                cp.start() if start else cp.wait()

            @pl.when(ch < _NCH - 1)
            def _():
                one(_ROWS)

            @pl.when(ch == _NCH - 1)
            def _():
                one(_TAIL)

        def do_write(t, ib, start):
            b, ch = task_parts(t)
            obase = b * _M + ch * _ROWS

            @pl.when(ch < _NCH - 1)
            def _():
                cp = pltpu.make_async_copy(
                    rows_v.at[ib], out_hbm.at[pl.ds(obase, _ROWS)], sem_w[ib])
                cp.start() if start else cp.wait()

            @pl.when(ch == _NCH - 1)
            def _():
                cp = pltpu.make_async_copy(
                    rows_v.at[ib].at[pl.ds(0, _TAIL)],
                    out_hbm.at[pl.ds(obase, _TAIL)], sem_w[ib])
                cp.start() if start else cp.wait()

        for kk in range(min(4, _KMAX)):
            t = task(kk)

            @pl.when(t < limit)
            def _(kk=kk, t=t):
                do_idx(kk, t, start=True)

        for kk in range(_NBUF - 1):
            t = task(kk)

            @pl.when(t < limit)
            def _(kk=kk, t=t):
                do_idx(kk, t, start=False)
                do_gathers(kk, t, start=True)
            if kk + 4 < _KMAX:
                tl = task(kk + 4)

                @pl.when(tl < limit)
                def _(kk=kk, tl=tl):
                    do_idx(kk + 4, tl, start=True)

        for kk in range(_KMAX):
            t = task(kk)

            if kk >= 1:
                @pl.when(task(kk - 1) < limit)
                def _(kk=kk):
                    do_write(task(kk - 1), (kk - 1) % _NBUF, start=False)

            if kk + _NBUF - 1 < _KMAX:
                tn = task(kk + _NBUF - 1)

                @pl.when(tn < limit)
                def _(kk=kk, tn=tn):
                    do_idx(kk + _NBUF - 1, tn, start=False)
                    do_gathers(kk + _NBUF - 1, tn, start=True)
                if kk + _NBUF + 3 < _KMAX:
                    tl = task(kk + _NBUF + 3)

                    @pl.when(tl < limit)
                    def _(kk=kk, tl=tl):
                        do_idx(kk + _NBUF + 3, tl, start=True)

            @pl.when(t < limit)
            def _(kk=kk, t=t):
                do_gathers(kk, t, start=False)
                do_write(t, kk % _NBUF, start=True)

        @pl.when(task(_KMAX - 1) < limit)
        def _():
            do_write(task(_KMAX - 1), (_KMAX - 1) % _NBUF, start=False)

    return k(table, idx)


def kernel(inputs, sampling_index):
    idx = sampling_index.astype(jnp.int32)
    out = _sc_gather(inputs, idx)
    return out.reshape(_B, _M, _C)
